# baseline (speedup 1.0000x reference)
import functools

import jax
import jax.numpy as jnp
from jax import lax
from jax.experimental import pallas as pl
from jax.experimental.pallas import tpu as pltpu
from jax.experimental.pallas import tpu_sc as plsc

N = 10000
E = 320000
D = 128
H = 64
Z = 32

NC = 2
NS = 16
NW = NC * NS
EPW = E // NW
C = 80
RCH = EPW // C
DEPTH = 3
TCH = RCH + 1
SV = TCH * C
NP_ = 10240
RPT = NP_ // NS

_mesh = plsc.VectorSubcoreMesh(core_axis_name="c", subcore_axis_name="s")


@functools.partial(
    pl.kernel,
    out_type=jax.ShapeDtypeStruct((NC, NP_, D), jnp.float32),
    mesh=_mesh,
    compiler_params=pltpu.CompilerParams(needs_layout_passes=False),
    scratch_types=[
        pltpu.VMEM((SV,), jnp.int32),
        *[pltpu.VMEM((C,), jnp.int32) for _ in range(DEPTH)],
        *[pltpu.VMEM((C,), jnp.float32) for _ in range(DEPTH)],
        *[pltpu.VMEM((C, D), jnp.float32) for _ in range(DEPTH)],
        pltpu.VMEM_SHARED((NP_, D), jnp.float32),
        pltpu.SemaphoreType.DMA,
        pltpu.SemaphoreType.DMA,
        *[pltpu.SemaphoreType.DMA for _ in range(DEPTH)],
        pltpu.SemaphoreType.DMA,
    ],
)
def _sc_aggregate(table, src, dst, attr, out, src_v, *rest):
    S = list(rest[0:DEPTH])
    A = list(rest[DEPTH:2 * DEPTH])
    R = list(rest[2 * DEPTH:3 * DEPTH])
    acc = rest[3 * DEPTH]
    sem_d = rest[3 * DEPTH + 1]
    sem_a = rest[3 * DEPTH + 2]
    sem_g = list(rest[3 * DEPTH + 3:4 * DEPTH + 3])
    sem_s = rest[4 * DEPTH + 3]

    cid = lax.axis_index("c")
    sid = lax.axis_index("s")
    wid = cid * NS + sid
    base_e = wid * EPW
    zero = jnp.zeros((16,), jnp.float32)
    izero = jnp.zeros((16,), jnp.int32)
    zrows = R[DEPTH - 1]

    def zrow(r, carry):
        for j in range(D // 16):
            zrows[r, pl.ds(j * 16, 16)] = zero
        return carry

    lax.fori_loop(0, C, zrow, None)
    for i in range(C // 16):
        S[DEPTH - 1][pl.ds(i * 16, 16)] = izero
    for i in range((SV - EPW) // 16):
        src_v[pl.ds(EPW + i * 16, 16)] = izero

    pltpu.sync_copy(src.at[pl.ds(base_e, EPW)], src_v.at[pl.ds(0, EPW)])

    def zcopy(b, carry):
        pltpu.sync_copy(zrows, acc.at[pl.ds(sid * RPT + b * C, C)])
        return carry

    lax.fori_loop(0, RPT // C, zcopy, None)
    plsc.subcore_barrier()

    pltpu.async_copy(zrows, acc.at[S[DEPTH - 1]], sem_s, add=True)
    pltpu.async_copy(dst.at[pl.ds(base_e, C)], S[0], sem_d)
    pltpu.async_copy(attr.at[pl.ds(base_e, C)], A[0], sem_a)
    pltpu.async_copy(table.at[src_v.at[pl.ds(0, C)]], R[0], sem_g[0])

    def step(k, rb):
        pb = (rb - 1) % DEPTH
        nb = (rb + 1) % DEPTH
        pltpu.make_async_copy(
            table.at[src_v.at[pl.ds(k * C, C)]], R[rb], sem_g[rb]).wait()

        @pl.when(k <= TCH - 2)
        def _():
            pltpu.async_copy(
                table.at[src_v.at[pl.ds((k + 1) * C, C)]],
                R[nb], sem_g[nb])

        @pl.when(k <= RCH - 2)
        def _():
            pltpu.async_copy(
                dst.at[pl.ds(base_e + (k + 1) * C, C)], S[nb], sem_d)
            pltpu.async_copy(
                attr.at[pl.ds(base_e + (k + 1) * C, C)], A[nb], sem_a)

        @pl.when(k <= RCH - 1)
        def _():
            pltpu.make_async_copy(
                dst.at[pl.ds(base_e + k * C, C)], S[rb], sem_d).wait()
            pltpu.make_async_copy(
                attr.at[pl.ds(base_e + k * C, C)], A[rb], sem_a).wait()

        for g in range(C // 16):
            a16 = A[rb][pl.ds(g * 16, 16)]
            for i in range(16):
                e = g * 16 + i
                av = jnp.full((16,), a16[i], jnp.float32)
                for j in range(D // 16):
                    R[rb][e, pl.ds(j * 16, 16)] = (
                        R[rb][e, pl.ds(j * 16, 16)] * av)

        pltpu.make_async_copy(R[pb], acc.at[S[pb]], sem_s).wait()

        @pl.when(k <= RCH - 1)
        def _():
            pltpu.async_copy(R[rb], acc.at[S[rb]], sem_s, add=True)

    def macro(m, carry):
        for b in range(DEPTH):
            step(DEPTH * m + b, b)
        return carry

    lax.fori_loop(0, TCH // DEPTH, macro, None)

    plsc.subcore_barrier()
    pltpu.sync_copy(acc.at[pl.ds(sid * RPT, RPT)],
                    out.at[cid, pl.ds(sid * RPT, RPT)])


BR = 1000


def _combine_body(x_ref, p_ref, o_ref):
    o_ref[...] = x_ref[...] + p_ref[0] + p_ref[1]


_combine = pl.pallas_call(
    _combine_body,
    grid=(N // BR,),
    in_specs=[
        pl.BlockSpec((BR, D), lambda i: (i, 0)),
        pl.BlockSpec((NC, BR, D), lambda i: (0, i, 0)),
    ],
    out_specs=pl.BlockSpec((BR, D), lambda i: (i, 0)),
    out_shape=jax.ShapeDtypeStruct((N, D), jnp.float32),
)


def _final_body(c_ref, p_ref, we0_ref, be0_ref, we1_ref, be1_ref,
                wd0_ref, bd0_ref, wd1_ref, bd1_ref, agg_ref, dec_ref):
    agg = c_ref[...] + p_ref[0] + p_ref[1]
    agg_ref[...] = agg
    h = jnp.maximum(
        jnp.dot(agg, we0_ref[...], preferred_element_type=jnp.float32)
        + be0_ref[...], 0.0)
    z = jnp.dot(h, we1_ref[...], preferred_element_type=jnp.float32) + be1_ref[...]
    h2 = jnp.maximum(
        jnp.dot(z, wd0_ref[...], preferred_element_type=jnp.float32)
        + bd0_ref[...], 0.0)
    dec_ref[...] = (
        jnp.dot(h2, wd1_ref[...], preferred_element_type=jnp.float32)
        + bd1_ref[...])


def _full_spec(shape):
    return pl.BlockSpec(shape, lambda i: tuple(0 for _ in shape))


_final = pl.pallas_call(
    _final_body,
    grid=(N // BR,),
    in_specs=[
        pl.BlockSpec((BR, D), lambda i: (i, 0)),
        pl.BlockSpec((NC, BR, D), lambda i: (0, i, 0)),
        _full_spec((D, H)),
        _full_spec((1, H)),
        _full_spec((H, Z)),
        _full_spec((1, Z)),
        _full_spec((Z, H)),
        _full_spec((1, H)),
        _full_spec((H, D)),
        _full_spec((1, D)),
    ],
    out_specs=[
        pl.BlockSpec((BR, D), lambda i: (i, 0)),
        pl.BlockSpec((BR, D), lambda i: (i, 0)),
    ],
    out_shape=[
        jax.ShapeDtypeStruct((N, D), jnp.float32),
        jax.ShapeDtypeStruct((N, D), jnp.float32),
    ],
)


def kernel(x, edge_index, edge_attr,
           W_e0, b_e0, W_e1, b_e1, W_d0, b_d0, W_d1, b_d1):
    src = edge_index[0]
    dst = edge_index[1]
    p1 = _sc_aggregate(x, src, dst, edge_attr)
    c1 = _combine(x, p1)
    p2 = _sc_aggregate(c1, src, dst, edge_attr)
    agg, dec = _final(
        c1, p2,
        W_e0.T, b_e0.reshape(1, H),
        W_e1.T, b_e1.reshape(1, Z),
        W_d0.T, b_d0.reshape(1, H),
        W_d1.T, b_d1.reshape(1, D),
    )
    return (agg, dec)


# speedup vs baseline: 1.0303x; 1.0303x over previous
import functools

import jax
import jax.numpy as jnp
from jax import lax
from jax.experimental import pallas as pl
from jax.experimental.pallas import tpu as pltpu
from jax.experimental.pallas import tpu_sc as plsc

N = 10000
E = 320000
D = 128
H = 64
Z = 32

NC = 2
NS = 16
NW = NC * NS
EPW = E // NW
C = 80
CHUNKS = EPW // C
NP_ = 10240
RPT = NP_ // NS
ZR = 128

_mesh = plsc.VectorSubcoreMesh(core_axis_name="c", subcore_axis_name="s")


@functools.partial(
    pl.kernel,
    out_type=jax.ShapeDtypeStruct((NC, NP_, D), jnp.float32),
    mesh=_mesh,
    compiler_params=pltpu.CompilerParams(needs_layout_passes=False),
    scratch_types=[
        pltpu.VMEM((EPW + C,), jnp.int32),
        pltpu.VMEM((EPW + C,), jnp.float32),
        pltpu.VMEM((C,), jnp.int32),
        pltpu.VMEM((C,), jnp.int32),
        pltpu.VMEM((C, D), jnp.float32),
        pltpu.VMEM((C, D), jnp.float32),
        pltpu.VMEM_SHARED((NP_, D), jnp.float32),
        pltpu.SemaphoreType.DMA,
        pltpu.SemaphoreType.DMA,
        pltpu.SemaphoreType.DMA,
    ],
)
def _sc_aggregate(table, src, dst, attr, out,
                  src_v, attr_v, sidx0, sidx1, rows0, rows1, acc,
                  sem_d, sem_g, sem_s):
    cid = lax.axis_index("c")
    sid = lax.axis_index("s")
    wid = cid * NS + sid
    base_e = wid * EPW
    R = [rows0, rows1]
    S = [sidx0, sidx1]
    zero = jnp.zeros((16,), jnp.float32)
    izero = jnp.zeros((16,), jnp.int32)

    def zrow(r, carry):
        for j in range(D // 16):
            rows0[r, pl.ds(j * 16, 16)] = zero
            rows1[r, pl.ds(j * 16, 16)] = zero
        return carry

    lax.fori_loop(0, C, zrow, None)
    for i in range(C // 16):
        sidx1[pl.ds(i * 16, 16)] = izero
        src_v[pl.ds(EPW + i * 16, 16)] = izero
        attr_v[pl.ds(EPW + i * 16, 16)] = zero

    pltpu.sync_copy(src.at[pl.ds(base_e, EPW)], src_v.at[pl.ds(0, EPW)])
    pltpu.sync_copy(attr.at[pl.ds(base_e, EPW)], attr_v.at[pl.ds(0, EPW)])

    def zcopy(b, carry):
        pltpu.sync_copy(rows0, acc.at[pl.ds(sid * RPT + b * C, C)])
        return carry

    lax.fori_loop(0, RPT // C, zcopy, None)
    plsc.subcore_barrier()

    pltpu.async_copy(rows1, acc.at[sidx1], sem_s, add=True)
    pltpu.async_copy(dst.at[pl.ds(base_e, C)], sidx0, sem_d)
    pltpu.async_copy(table.at[src_v.at[pl.ds(0, C)]], rows0, sem_g)

    def step(k, rb):
        nrb = 1 - rb
        pltpu.make_async_copy(R[nrb], acc.at[S[nrb]], sem_s).wait()
        pltpu.make_async_copy(
            table.at[src_v.at[pl.ds(k * C, C)]], R[rb], sem_g).wait()

        @pl.when(k <= CHUNKS - 1)
        def _():
            pltpu.async_copy(
                table.at[src_v.at[pl.ds((k + 1) * C, C)]], R[nrb], sem_g)

        @pl.when(k <= CHUNKS - 1)
        def _():
            pltpu.make_async_copy(
                dst.at[pl.ds(base_e + k * C, C)], S[rb], sem_d).wait()

        @pl.when(k <= CHUNKS - 2)
        def _():
            pltpu.async_copy(
                dst.at[pl.ds(base_e + (k + 1) * C, C)], S[nrb], sem_d)

        for g in range(C // 16):
            a16 = attr_v[pl.ds(k * C + g * 16, 16)]
            for i in range(16):
                e = g * 16 + i
                av = jnp.full((16,), a16[i], jnp.float32)
                for j in range(D // 16):
                    R[rb][e, pl.ds(j * 16, 16)] = (
                        R[rb][e, pl.ds(j * 16, 16)] * av)

        pltpu.async_copy(R[rb], acc.at[S[rb]], sem_s, add=True)

    def macro(m, carry):
        step(2 * m, 0)
        step(2 * m + 1, 1)
        return carry

    lax.fori_loop(0, (CHUNKS + 1) // 2, macro, None)

    pltpu.make_async_copy(rows1, acc.at[sidx1], sem_s).wait()
    plsc.subcore_barrier()
    pltpu.sync_copy(acc.at[pl.ds(sid * RPT, RPT)],
                    out.at[cid, pl.ds(sid * RPT, RPT)])


BR = 1000


def _combine_body(x_ref, p_ref, o_ref):
    o_ref[...] = x_ref[...] + p_ref[0] + p_ref[1]


_combine = pl.pallas_call(
    _combine_body,
    grid=(N // BR,),
    in_specs=[
        pl.BlockSpec((BR, D), lambda i: (i, 0)),
        pl.BlockSpec((NC, BR, D), lambda i: (0, i, 0)),
    ],
    out_specs=pl.BlockSpec((BR, D), lambda i: (i, 0)),
    out_shape=jax.ShapeDtypeStruct((N, D), jnp.float32),
)


def _final_body(c_ref, p_ref, we0_ref, be0_ref, we1_ref, be1_ref,
                wd0_ref, bd0_ref, wd1_ref, bd1_ref, agg_ref, dec_ref):
    agg = c_ref[...] + p_ref[0] + p_ref[1]
    agg_ref[...] = agg
    h = jnp.maximum(
        jnp.dot(agg, we0_ref[...], preferred_element_type=jnp.float32)
        + be0_ref[...], 0.0)
    z = jnp.dot(h, we1_ref[...], preferred_element_type=jnp.float32) + be1_ref[...]
    h2 = jnp.maximum(
        jnp.dot(z, wd0_ref[...], preferred_element_type=jnp.float32)
        + bd0_ref[...], 0.0)
    dec_ref[...] = (
        jnp.dot(h2, wd1_ref[...], preferred_element_type=jnp.float32)
        + bd1_ref[...])


def _full_spec(shape):
    return pl.BlockSpec(shape, lambda i: tuple(0 for _ in shape))


_final = pl.pallas_call(
    _final_body,
    grid=(N // BR,),
    in_specs=[
        pl.BlockSpec((BR, D), lambda i: (i, 0)),
        pl.BlockSpec((NC, BR, D), lambda i: (0, i, 0)),
        _full_spec((D, H)),
        _full_spec((1, H)),
        _full_spec((H, Z)),
        _full_spec((1, Z)),
        _full_spec((Z, H)),
        _full_spec((1, H)),
        _full_spec((H, D)),
        _full_spec((1, D)),
    ],
    out_specs=[
        pl.BlockSpec((BR, D), lambda i: (i, 0)),
        pl.BlockSpec((BR, D), lambda i: (i, 0)),
    ],
    out_shape=[
        jax.ShapeDtypeStruct((N, D), jnp.float32),
        jax.ShapeDtypeStruct((N, D), jnp.float32),
    ],
)


def kernel(x, edge_index, edge_attr,
           W_e0, b_e0, W_e1, b_e1, W_d0, b_d0, W_d1, b_d1):
    src = edge_index[0]
    dst = edge_index[1]
    p1 = _sc_aggregate(x, src, dst, edge_attr)
    c1 = _combine(x, p1)
    p2 = _sc_aggregate(c1, src, dst, edge_attr)
    agg, dec = _final(
        c1, p2,
        W_e0.T, b_e0.reshape(1, H),
        W_e1.T, b_e1.reshape(1, Z),
        W_d0.T, b_d0.reshape(1, H),
        W_d1.T, b_d1.reshape(1, D),
    )
    return (agg, dec)
